# baseline (speedup 1.0000x reference)
import functools

import jax
import jax.numpy as jnp
from jax.experimental import pallas as pl
from jax.experimental.pallas import tpu as pltpu

_EPS = 1e-8
_LANE = 128


def _cdiv(a, b):
    return -(-a // b)


def _accumulate(s1_ref, s2_ref, acc_ref, tb, chunk, n_chunks, valid_len):
    z = jnp.zeros((tb, chunk), jnp.float32)
    m1, m2, v11, v22, v12 = z, z, z, z, z
    for c in range(n_chunks):
        off = c * chunk
        x1 = s1_ref[:, off:off + chunk]
        x2 = s2_ref[:, off:off + chunk]
        if off + chunk > valid_len:
            lane = jax.lax.broadcasted_iota(jnp.int32, (tb, chunk), 1)
            keep = lane < (valid_len - off)
            x1 = jnp.where(keep, x1, 0.0)
            x2 = jnp.where(keep, x2, 0.0)
        m1 = m1 + x1
        m2 = m2 + x2
        v11 = v11 + x1 * x1
        v22 = v22 + x2 * x2
        v12 = v12 + x1 * x2
    acc_ref[0] += m1
    acc_ref[1] += m2
    acc_ref[2] += v11
    acc_ref[3] += v22
    acc_ref[4] += v12


def _sdr_kernel(s1_ref, s2_ref, out_ref, acc_ref, *,
                length, block_l, chunk, n_l, eps):
    j = pl.program_id(1)
    tb = out_ref.shape[0]

    @pl.when(j == 0)
    def _init():
        acc_ref[...] = jnp.zeros_like(acc_ref)

    tail = length - (n_l - 1) * block_l
    if n_l == 1:
        _accumulate(s1_ref, s2_ref, acc_ref, tb, chunk,
                    _cdiv(tail, chunk), tail)
    elif tail == block_l:
        _accumulate(s1_ref, s2_ref, acc_ref, tb, chunk,
                    block_l // chunk, block_l)
    else:
        @pl.when(j < n_l - 1)
        def _full():
            _accumulate(s1_ref, s2_ref, acc_ref, tb, chunk,
                        block_l // chunk, block_l)

        @pl.when(j == n_l - 1)
        def _tail():
            _accumulate(s1_ref, s2_ref, acc_ref, tb, chunk,
                        _cdiv(tail, chunk), tail)

    @pl.when(j == n_l - 1)
    def _epilogue():
        s1m = jnp.sum(acc_ref[0], axis=-1, keepdims=True)
        s2m = jnp.sum(acc_ref[1], axis=-1, keepdims=True)
        p11 = jnp.sum(acc_ref[2], axis=-1, keepdims=True)
        p22 = jnp.sum(acc_ref[3], axis=-1, keepdims=True)
        p12 = jnp.sum(acc_ref[4], axis=-1, keepdims=True)

        inv_len = jnp.float32(1.0 / length)
        c11 = p11 - s1m * s1m * inv_len
        c22 = p22 - s2m * s2m * inv_len
        c12 = p12 - s1m * s2m * inv_len

        alpha = c12 / (c22 + eps)
        target = alpha * alpha * c22
        noise = c11 - 2.0 * alpha * c12 + target
        out_ref[...] = -10.0 * jnp.log10(target / (noise + eps) + eps)


def _pick_tiles(B, L):
    tb = 32 if B % 32 == 0 else (8 if B % 8 == 0 else B)
    chunk = _LANE
    target = 25 * chunk
    tl = None
    for cand in range(target, 0, -chunk):
        if L % cand == 0:
            tl = cand
            break
    if tl is None:
        tl = min(target, _cdiv(L, chunk) * chunk)
    return tb, tl, chunk


def kernel(s1, s2):
    assert s1.ndim == 2 and s1.shape == s2.shape
    B, L = s1.shape
    tb, tl, chunk = _pick_tiles(B, L)
    n_b = _cdiv(B, tb)
    n_l = _cdiv(L, tl)

    body = functools.partial(
        _sdr_kernel,
        length=L, block_l=tl, chunk=chunk, n_l=n_l, eps=_EPS,
    )

    neg_snr = pl.pallas_call(
        body,
        out_shape=jax.ShapeDtypeStruct((n_b * tb, 1), jnp.float32),
        grid=(n_b, n_l),
        in_specs=[
            pl.BlockSpec((tb, tl), lambda i, j: (i, j)),
            pl.BlockSpec((tb, tl), lambda i, j: (i, j)),
        ],
        out_specs=pl.BlockSpec((tb, 1), lambda i, j: (i, 0)),
        scratch_shapes=[pltpu.VMEM((5, tb, chunk), jnp.float32)],
        compiler_params=pltpu.CompilerParams(
            dimension_semantics=("parallel", "arbitrary"),
            vmem_limit_bytes=48 * 1024 * 1024,
        ),
    )(s1, s2)

    return jnp.mean(neg_snr[:B])


# speedup vs baseline: 2.1300x; 2.1300x over previous
import functools

import jax
import jax.numpy as jnp
from jax.experimental import pallas as pl
from jax.experimental.pallas import tpu as pltpu

_EPS = 1e-8
_LANE = 128


def _cdiv(a, b):
    return -(-a // b)


def _accumulate(s1_ref, s2_ref, acc_ref, tb, chunk, n_chunks, valid_len):
    z = jnp.zeros((tb, chunk), jnp.float32)
    m1, m2, v11, v22, v12 = z, z, z, z, z
    for c in range(n_chunks):
        off = c * chunk
        x1 = s1_ref[:, off:off + chunk]
        x2 = s2_ref[:, off:off + chunk]
        if off + chunk > valid_len:
            lane = jax.lax.broadcasted_iota(jnp.int32, (tb, chunk), 1)
            keep = lane < (valid_len - off)
            x1 = jnp.where(keep, x1, 0.0)
            x2 = jnp.where(keep, x2, 0.0)
        m1 = m1 + x1
        m2 = m2 + x2
        v11 = v11 + x1 * x1
        v22 = v22 + x2 * x2
        v12 = v12 + x1 * x2
    acc_ref[0] += m1
    acc_ref[1] += m2
    acc_ref[2] += v11
    acc_ref[3] += v22
    acc_ref[4] += v12


def _sdr_kernel(s1_ref, s2_ref, out_ref, acc_ref, *,
                length, block_l, chunk, n_l, eps):
    j = pl.program_id(1)
    tb = out_ref.shape[0]

    @pl.when(j == 0)
    def _init():
        acc_ref[...] = jnp.zeros_like(acc_ref)

    tail = length - (n_l - 1) * block_l
    if n_l == 1:
        _accumulate(s1_ref, s2_ref, acc_ref, tb, chunk,
                    _cdiv(tail, chunk), tail)
    elif tail == block_l:
        _accumulate(s1_ref, s2_ref, acc_ref, tb, chunk,
                    block_l // chunk, block_l)
    else:
        @pl.when(j < n_l - 1)
        def _full():
            _accumulate(s1_ref, s2_ref, acc_ref, tb, chunk,
                        block_l // chunk, block_l)

        @pl.when(j == n_l - 1)
        def _tail():
            _accumulate(s1_ref, s2_ref, acc_ref, tb, chunk,
                        _cdiv(tail, chunk), tail)

    @pl.when(j == n_l - 1)
    def _epilogue():
        s1m = jnp.sum(acc_ref[0], axis=-1, keepdims=True)
        s2m = jnp.sum(acc_ref[1], axis=-1, keepdims=True)
        p11 = jnp.sum(acc_ref[2], axis=-1, keepdims=True)
        p22 = jnp.sum(acc_ref[3], axis=-1, keepdims=True)
        p12 = jnp.sum(acc_ref[4], axis=-1, keepdims=True)

        inv_len = jnp.float32(1.0 / length)
        c11 = p11 - s1m * s1m * inv_len
        c22 = p22 - s2m * s2m * inv_len
        c12 = p12 - s1m * s2m * inv_len

        alpha = c12 / (c22 + eps)
        target = alpha * alpha * c22
        noise = c11 - 2.0 * alpha * c12 + target
        out_ref[...] = -10.0 * jnp.log10(target / (noise + eps) + eps)


def _pick_tiles(B, L):
    tb = 32 if B % 32 == 0 else (8 if B % 8 == 0 else B)
    chunk = _LANE
    target = 125 * chunk
    tl = None
    for cand in range(target, 0, -chunk):
        if L % cand == 0:
            tl = cand
            break
    if tl is None:
        tl = min(target, _cdiv(L, chunk) * chunk)
    return tb, tl, chunk


def kernel(s1, s2):
    assert s1.ndim == 2 and s1.shape == s2.shape
    B, L = s1.shape
    tb, tl, chunk = _pick_tiles(B, L)
    n_b = _cdiv(B, tb)
    n_l = _cdiv(L, tl)

    body = functools.partial(
        _sdr_kernel,
        length=L, block_l=tl, chunk=chunk, n_l=n_l, eps=_EPS,
    )

    neg_snr = pl.pallas_call(
        body,
        out_shape=jax.ShapeDtypeStruct((n_b * tb, 1), jnp.float32),
        grid=(n_b, n_l),
        in_specs=[
            pl.BlockSpec((tb, tl), lambda i, j: (i, j)),
            pl.BlockSpec((tb, tl), lambda i, j: (i, j)),
        ],
        out_specs=pl.BlockSpec((tb, 1), lambda i, j: (i, 0)),
        scratch_shapes=[pltpu.VMEM((5, tb, chunk), jnp.float32)],
        compiler_params=pltpu.CompilerParams(
            dimension_semantics=("parallel", "arbitrary"),
            vmem_limit_bytes=48 * 1024 * 1024,
        ),
    )(s1, s2)

    return jnp.mean(neg_snr[:B])


# speedup vs baseline: 2.4186x; 1.1355x over previous
import functools

import jax
import jax.numpy as jnp
from jax.experimental import pallas as pl
from jax.experimental.pallas import tpu as pltpu

_EPS = 1e-8
_LANE = 128


def _cdiv(a, b):
    return -(-a // b)


def _accumulate(s1_ref, s2_ref, acc_ref, tb, chunk, n_chunks, valid_len):
    z = jnp.zeros((tb, chunk), jnp.float32)
    m1, m2, v11, v22, v12 = z, z, z, z, z
    for c in range(n_chunks):
        off = c * chunk
        x1 = s1_ref[:, off:off + chunk]
        x2 = s2_ref[:, off:off + chunk]
        if off + chunk > valid_len:
            lane = jax.lax.broadcasted_iota(jnp.int32, (tb, chunk), 1)
            keep = lane < (valid_len - off)
            x1 = jnp.where(keep, x1, 0.0)
            x2 = jnp.where(keep, x2, 0.0)
        m1 = m1 + x1
        m2 = m2 + x2
        v11 = v11 + x1 * x1
        v22 = v22 + x2 * x2
        v12 = v12 + x1 * x2
    acc_ref[0] += m1
    acc_ref[1] += m2
    acc_ref[2] += v11
    acc_ref[3] += v22
    acc_ref[4] += v12


def _sdr_kernel(s1_ref, s2_ref, out_ref, acc_ref, *,
                length, block_l, chunk, n_l, eps):
    j = pl.program_id(1)
    tb = out_ref.shape[0]

    @pl.when(j == 0)
    def _init():
        acc_ref[...] = jnp.zeros_like(acc_ref)

    tail = length - (n_l - 1) * block_l
    if n_l == 1:
        _accumulate(s1_ref, s2_ref, acc_ref, tb, chunk,
                    _cdiv(tail, chunk), tail)
    elif tail == block_l:
        _accumulate(s1_ref, s2_ref, acc_ref, tb, chunk,
                    block_l // chunk, block_l)
    else:
        @pl.when(j < n_l - 1)
        def _full():
            _accumulate(s1_ref, s2_ref, acc_ref, tb, chunk,
                        block_l // chunk, block_l)

        @pl.when(j == n_l - 1)
        def _tail():
            _accumulate(s1_ref, s2_ref, acc_ref, tb, chunk,
                        _cdiv(tail, chunk), tail)

    @pl.when(j == n_l - 1)
    def _epilogue():
        s1m = jnp.sum(acc_ref[0], axis=-1, keepdims=True)
        s2m = jnp.sum(acc_ref[1], axis=-1, keepdims=True)
        p11 = jnp.sum(acc_ref[2], axis=-1, keepdims=True)
        p22 = jnp.sum(acc_ref[3], axis=-1, keepdims=True)
        p12 = jnp.sum(acc_ref[4], axis=-1, keepdims=True)

        inv_len = jnp.float32(1.0 / length)
        c11 = p11 - s1m * s1m * inv_len
        c22 = p22 - s2m * s2m * inv_len
        c12 = p12 - s1m * s2m * inv_len

        alpha = c12 / (c22 + eps)
        target = alpha * alpha * c22
        noise = c11 - 2.0 * alpha * c12 + target
        out_ref[...] = -10.0 * jnp.log10(target / (noise + eps) + eps)


def _pick_tiles(B, L):
    tb = 32 if B % 32 == 0 else (8 if B % 8 == 0 else B)
    chunk = _LANE
    target = 250 * chunk
    tl = None
    for cand in range(target, 0, -chunk):
        if L % cand == 0:
            tl = cand
            break
    if tl is None:
        tl = min(target, _cdiv(L, chunk) * chunk)
    return tb, tl, chunk


def kernel(s1, s2):
    assert s1.ndim == 2 and s1.shape == s2.shape
    B, L = s1.shape
    tb, tl, chunk = _pick_tiles(B, L)
    n_b = _cdiv(B, tb)
    n_l = _cdiv(L, tl)

    body = functools.partial(
        _sdr_kernel,
        length=L, block_l=tl, chunk=chunk, n_l=n_l, eps=_EPS,
    )

    neg_snr = pl.pallas_call(
        body,
        out_shape=jax.ShapeDtypeStruct((n_b * tb, 1), jnp.float32),
        grid=(n_b, n_l),
        in_specs=[
            pl.BlockSpec((tb, tl), lambda i, j: (i, j)),
            pl.BlockSpec((tb, tl), lambda i, j: (i, j)),
        ],
        out_specs=pl.BlockSpec((tb, 1), lambda i, j: (i, 0)),
        scratch_shapes=[pltpu.VMEM((5, tb, chunk), jnp.float32)],
        compiler_params=pltpu.CompilerParams(
            dimension_semantics=("parallel", "arbitrary"),
            vmem_limit_bytes=48 * 1024 * 1024,
        ),
    )(s1, s2)

    return jnp.mean(neg_snr[:B])
